# Initial kernel scaffold; baseline (speedup 1.0000x reference)
import jax
import jax.numpy as jnp
from jax.experimental import pallas as pl


def kernel(x, edge_index, W1, b1, W2, b2):
    raise NotImplementedError("write your pallas kernel here")


# speedup vs baseline: 9.0920x; 9.0920x over previous
import functools

import jax
import jax.numpy as jnp
from jax import lax
from jax.experimental import pallas as pl
from jax.experimental.pallas import tpu as pltpu
from jax.experimental.pallas import tpu_sc as plsc

N = 10000
NACC = 10112
TRASH = 10080
NCORES = 2
NSUB = 16
CHUNK = 128


def _sc_mesh():
    return plsc.VectorSubcoreMesh(
        core_axis_name="c", subcore_axis_name="s",
        num_cores=NCORES, num_subcores=NSUB)


def _deg_kernel(dst_pad, ones, zeros):
    e_pad = dst_pad.shape[0]
    per_tile = e_pad // (NCORES * NSUB)
    nchunks = per_tile // CHUNK

    @functools.partial(
        pl.kernel,
        out_type=jax.ShapeDtypeStruct((NCORES * NACC, 128), jnp.float32),
        mesh=_sc_mesh(),
        scratch_types=[
            pltpu.VMEM((CHUNK,), jnp.int32),
            pltpu.VMEM((CHUNK, 128), jnp.float32),
            pltpu.VMEM_SHARED((NACC, 128), jnp.float32),
        ],
    )
    def k(dst_hbm, ones_hbm, zeros_hbm, out_hbm, didx_v, ones_v, acc_sh):
        c = lax.axis_index("c")
        s = lax.axis_index("s")
        init_rows = NACC // NSUB
        pltpu.sync_copy(zeros_hbm.at[pl.ds(s * init_rows, init_rows)],
                        acc_sh.at[pl.ds(s * init_rows, init_rows)])
        pltpu.sync_copy(ones_hbm, ones_v)
        plsc.subcore_barrier()

        tid = c * NSUB + s
        base0 = tid * per_tile

        @pl.loop(0, nchunks)
        def _(j):
            pltpu.sync_copy(dst_hbm.at[pl.ds(base0 + j * CHUNK, CHUNK)],
                            didx_v)
            pltpu.sync_copy(ones_v, acc_sh.at[didx_v], add=True)

        plsc.subcore_barrier()
        pltpu.sync_copy(
            acc_sh.at[pl.ds(s * init_rows, init_rows)],
            out_hbm.at[pl.ds(c * NACC + s * init_rows, init_rows)])

    return k(dst_pad, ones, zeros)


def _spmm_kernel(table, srcoff_flat, dst_pad, zeros, dblk):
    e_pad = dst_pad.shape[0]
    per_sub = e_pad // NSUB
    nchunks = per_sub // CHUNK

    @functools.partial(
        pl.kernel,
        out_type=jax.ShapeDtypeStruct((NCORES * NACC, dblk), jnp.float32),
        mesh=_sc_mesh(),
        scratch_types=[
            pltpu.VMEM((CHUNK,), jnp.int32),
            pltpu.VMEM((CHUNK,), jnp.int32),
            pltpu.VMEM((CHUNK, dblk), jnp.float32),
            pltpu.VMEM_SHARED((NACC, dblk), jnp.float32),
            pltpu.SemaphoreType.DMA,
        ],
    )
    def k(table_hbm, srcoff_hbm, dst_hbm, zeros_hbm, out_hbm,
          sidx_v, didx_v, rows_v, acc_sh, sem):
        c = lax.axis_index("c")
        s = lax.axis_index("s")
        init_rows = NACC // NSUB
        pltpu.sync_copy(zeros_hbm.at[pl.ds(s * init_rows, init_rows)],
                        acc_sh.at[pl.ds(s * init_rows, init_rows)])
        plsc.subcore_barrier()

        base0 = s * per_sub
        srcbase = c * e_pad + base0

        @pl.loop(0, nchunks)
        def _(j):
            pltpu.sync_copy(srcoff_hbm.at[pl.ds(srcbase + j * CHUNK, CHUNK)],
                            sidx_v)
            pltpu.sync_copy(dst_hbm.at[pl.ds(base0 + j * CHUNK, CHUNK)],
                            didx_v)
            pltpu.async_copy(table_hbm.at[sidx_v], rows_v, sem).wait()
            pltpu.sync_copy(rows_v, acc_sh.at[didx_v], add=True)

        plsc.subcore_barrier()
        pltpu.sync_copy(
            acc_sh.at[pl.ds(s * init_rows, init_rows)],
            out_hbm.at[pl.ds(c * NACC + s * init_rows, init_rows)])

    return k(table, srcoff_flat, dst_pad, zeros)


def _spmm_edgesplit_kernel(table, src_pad, dst_pad, zeros):
    e_pad = dst_pad.shape[0]
    per_tile = e_pad // (NCORES * NSUB)
    nchunks = per_tile // CHUNK

    @functools.partial(
        pl.kernel,
        out_type=jax.ShapeDtypeStruct((NCORES * NACC, 128), jnp.float32),
        mesh=_sc_mesh(),
        scratch_types=[
            pltpu.VMEM((CHUNK,), jnp.int32),
            pltpu.VMEM((CHUNK,), jnp.int32),
            pltpu.VMEM((CHUNK, 128), jnp.float32),
            pltpu.VMEM_SHARED((NACC, 128), jnp.float32),
            pltpu.SemaphoreType.DMA,
        ],
    )
    def k(table_hbm, src_hbm, dst_hbm, zeros_hbm, out_hbm,
          sidx_v, didx_v, rows_v, acc_sh, sem):
        c = lax.axis_index("c")
        s = lax.axis_index("s")
        init_rows = NACC // NSUB
        pltpu.sync_copy(zeros_hbm.at[pl.ds(s * init_rows, init_rows)],
                        acc_sh.at[pl.ds(s * init_rows, init_rows)])
        plsc.subcore_barrier()

        tid = c * NSUB + s
        base0 = tid * per_tile

        @pl.loop(0, nchunks)
        def _(j):
            pltpu.sync_copy(src_hbm.at[pl.ds(base0 + j * CHUNK, CHUNK)],
                            sidx_v)
            pltpu.sync_copy(dst_hbm.at[pl.ds(base0 + j * CHUNK, CHUNK)],
                            didx_v)
            pltpu.async_copy(table_hbm.at[sidx_v], rows_v, sem).wait()
            pltpu.sync_copy(rows_v, acc_sh.at[didx_v], add=True)

        plsc.subcore_barrier()
        pltpu.sync_copy(
            acc_sh.at[pl.ds(s * init_rows, init_rows)],
            out_hbm.at[pl.ds(c * NACC + s * init_rows, init_rows)])

    return k(table, src_pad, dst_pad, zeros)


_HI = jax.lax.Precision.HIGHEST


def _dot(a, b):
    return jax.lax.dot_general(a, b, (((1,), (0,)), ((), ())),
                               precision=_HI,
                               preferred_element_type=jnp.float32)


def _xw1_body(x_ref, w1_ref, out_ref):
    x = x_ref[...]
    out_ref[0:N, :] = _dot(x, w1_ref[:, 0:128])
    out_ref[N:2 * N, :] = _dot(x, w1_ref[:, 128:256])


def _dinv(degp_ref):
    deg = degp_ref[0:N, 0:1] + degp_ref[NACC:NACC + N, 0:1] + 1.0
    return jax.lax.rsqrt(deg)


def _scale_body(xw_ref, degp_ref, out_ref):
    dinv = _dinv(degp_ref)
    out_ref[0:N, :] = dinv * xw_ref[0:N, :]
    out_ref[N:2 * N, :] = dinv * xw_ref[N:2 * N, :]


_MIDB = 2000


def _mid_body(z1_ref, y1_ref, degp_ref, b1_ref, w2_ref, out_ref):
    deg = degp_ref[0, :, 0:1] + degp_ref[1, :, 0:1] + 1.0
    dinv = jax.lax.rsqrt(deg)
    h0 = jnp.maximum(
        dinv * (z1_ref[0, :, :] + y1_ref[0, :, :]) + b1_ref[0:1, 0:128], 0.0)
    h1 = jnp.maximum(
        dinv * (z1_ref[1, :, :] + y1_ref[1, :, :]) + b1_ref[0:1, 128:256], 0.0)
    xw2 = _dot(h0, w2_ref[0:128, :]) + _dot(h1, w2_ref[128:256, :])
    out_ref[...] = dinv * xw2


def _mid_call(z1, y1, degp, b1, w2):
    return pl.pallas_call(
        _mid_body,
        grid=(N // _MIDB,),
        in_specs=[
            pl.BlockSpec((2, _MIDB, 128), lambda i: (0, i, 0)),
            pl.BlockSpec((2, _MIDB, 128), lambda i: (0, i, 0)),
            pl.BlockSpec((2, _MIDB, 128), lambda i: (0, i, 0)),
            pl.BlockSpec((1, 256), lambda i: (0, 0)),
            pl.BlockSpec((256, 128), lambda i: (0, 0)),
        ],
        out_specs=pl.BlockSpec((_MIDB, 128), lambda i: (i, 0)),
        out_shape=jax.ShapeDtypeStruct((N, 128), jnp.float32),
    )(z1.reshape(2, NACC, 128), y1.reshape(2, N, 128),
      degp.reshape(2, NACC, 128), b1, w2)


def _final_body(z2_ref, y2_ref, degp_ref, b2_ref, out_ref):
    dinv = _dinv(degp_ref)
    out_ref[...] = dinv * (z2_ref[0:N, :] + z2_ref[NACC:NACC + N, :]
                           + y2_ref[...]) + b2_ref[...]


def _tc_call(body, out_shape, *args):
    return pl.pallas_call(
        body,
        out_shape=jax.ShapeDtypeStruct(out_shape, jnp.float32),
    )(*args)


def kernel(x, edge_index, W1, b1, W2, b2):
    e = edge_index.shape[1]
    e_pad = ((e + NCORES * NSUB * CHUNK - 1)
             // (NCORES * NSUB * CHUNK)) * (NCORES * NSUB * CHUNK)
    pad = e_pad - e

    src = edge_index[0].astype(jnp.int32)
    dst = edge_index[1].astype(jnp.int32)
    dst_pad = jnp.concatenate(
        [dst, jnp.full((pad,), TRASH, jnp.int32)])
    srcoff = jnp.concatenate([
        src, jnp.zeros((pad,), jnp.int32),
        src + N, jnp.full((pad,), N, jnp.int32),
    ])

    ones128 = jnp.ones((CHUNK, 128), jnp.float32)
    zeros128 = jnp.zeros((NACC, 128), jnp.float32)

    degp = _deg_kernel(dst_pad, ones128, zeros128)

    xw1 = _tc_call(_xw1_body, (2 * N, 128), x, W1)
    y1 = _tc_call(_scale_body, (2 * N, 128), xw1, degp)

    z1 = _spmm_kernel(y1, srcoff, dst_pad, zeros128, 128)

    y2 = _mid_call(z1, y1, degp, b1.reshape(1, -1), W2)

    z2 = _spmm_edgesplit_kernel(y2, srcoff, dst_pad, zeros128)

    out = _tc_call(_final_body, (N, 128), z2, y2, degp,
                   b2.reshape(1, -1))
    return out
